# baseline (speedup 1.0000x reference)
import jax
import jax.numpy as jnp
from jax.experimental import pallas as pl

_BS = 8
_BB = 4096


def _mm_kernel(x_ref, wt_ref, o_ref):
    wt = wt_ref[...]
    for s in range(_BS):
        xs = x_ref[:, s, :]
        o_ref[s] = jax.lax.dot_general(
            wt, xs,
            dimension_numbers=(((1,), (0,)), ((), ())),
            preferred_element_type=jnp.float32,
            precision=jax.lax.Precision.DEFAULT,
        )


def kernel(inputs, embeddingRST):
    B, S, K = inputs.shape
    N = embeddingRST.shape[1]
    x_t = jnp.transpose(inputs, (2, 1, 0))
    w_t = embeddingRST.T
    out_t = pl.pallas_call(
        _mm_kernel,
        grid=(S // _BS, B // _BB),
        in_specs=[
            pl.BlockSpec((K, _BS, _BB), lambda si, bi: (0, si, bi)),
            pl.BlockSpec((N, K), lambda si, bi: (0, 0)),
        ],
        out_specs=pl.BlockSpec((_BS, N, _BB), lambda si, bi: (si, 0, bi)),
        out_shape=jax.ShapeDtypeStruct((S, N, B), jnp.float32),
    )(x_t, w_t)
    return jnp.transpose(out_t, (2, 0, 1))


# speedup vs baseline: 1.0264x; 1.0264x over previous
import jax
import jax.numpy as jnp
from jax.experimental import pallas as pl

_BS = 8
_BB = 8192


def _mm_kernel(x_ref, wt_ref, o_ref):
    wt = wt_ref[...]
    for s in range(_BS):
        xs = x_ref[:, s, :]
        o_ref[s] = jax.lax.dot_general(
            wt, xs,
            dimension_numbers=(((1,), (0,)), ((), ())),
            preferred_element_type=jnp.float32,
            precision=jax.lax.Precision.DEFAULT,
        )


def kernel(inputs, embeddingRST):
    B, S, K = inputs.shape
    N = embeddingRST.shape[1]
    x_t = jnp.transpose(inputs, (2, 1, 0))
    w_t = embeddingRST.T
    out_t = pl.pallas_call(
        _mm_kernel,
        grid=(S // _BS, B // _BB),
        in_specs=[
            pl.BlockSpec((K, _BS, _BB), lambda si, bi: (0, si, bi)),
            pl.BlockSpec((N, K), lambda si, bi: (0, 0)),
        ],
        out_specs=pl.BlockSpec((_BS, N, _BB), lambda si, bi: (si, 0, bi)),
        out_shape=jax.ShapeDtypeStruct((S, N, B), jnp.float32),
    )(x_t, w_t)
    return jnp.transpose(out_t, (2, 0, 1))
